# baseline (speedup 1.0000x reference)
import functools

import jax
import jax.numpy as jnp
from jax import lax
from jax.experimental import pallas as pl
from jax.experimental.pallas import tpu as pltpu
from jax.experimental.pallas import tpu_sc as plsc


def _sigma_inv_tc(weight, u2, blk):
    v_rows, d = weight.shape
    nsteps = v_rows // blk

    def body(w_ref, u_ref, o_ref, g_acc, t_acc):
        i = pl.program_id(0)

        @pl.when(i == 0)
        def _init():
            g_acc[...] = jnp.zeros_like(g_acc)
            t_acc[...] = jnp.zeros_like(t_acc)

        w = w_ref[...]
        g_acc[...] += lax.dot_general(
            w, w, (((0,), (0,)), ((), ())), preferred_element_type=jnp.float32)
        t_acc[...] += lax.dot_general(
            u_ref[0], w, (((1,), (0,)), ((), ())),
            preferred_element_type=jnp.float32)

        @pl.when(i == nsteps - 1)
        def _finish():
            g = g_acc[...]
            t = t_acc[...]
            q = lax.dot_general(
                t, g, (((1,), (0,)), ((), ())),
                preferred_element_type=jnp.float32)
            s2 = jnp.sum(q * t)
            n2 = jnp.sum(t * t)
            sigma = jnp.sqrt(s2) / (jnp.sqrt(n2) + 1e-12)
            o_ref[0, 0] = 1.0 / sigma

    return pl.pallas_call(
        body,
        grid=(nsteps,),
        in_specs=[
            pl.BlockSpec((blk, d), lambda i: (i, 0)),
            pl.BlockSpec((1, 1, blk), lambda i: (i, 0, 0)),
        ],
        out_specs=pl.BlockSpec((1, 1), lambda i: (0, 0),
                               memory_space=pltpu.SMEM),
        out_shape=jax.ShapeDtypeStruct((1, 1), jnp.float32),
        scratch_shapes=[
            pltpu.VMEM((d, d), jnp.float32),
            pltpu.VMEM((1, d), jnp.float32),
        ],
        compiler_params=pltpu.CompilerParams(
            dimension_semantics=("arbitrary",)),
    )(weight, u2)


def _sc_workers():
    try:
        info = plsc.get_sparse_core_info()
        return info.num_cores, info.num_subcores
    except Exception:
        return 2, 16


def _gather_scale_sc(table, idx2, sinv16):
    nc, ns = _sc_workers()
    nw = nc * ns
    n_rows = idx2.shape[0]
    v_rows, d = table.shape
    rpw = n_rows // nw
    assert n_rows % nw == 0 and rpw % 2 == 0 and d % 16 == 0
    bt = n_rows * 128
    idx3 = idx2.reshape(nw, rpw, 128)
    mesh = plsc.VectorSubcoreMesh(core_axis_name="c", subcore_axis_name="s")

    @functools.partial(
        pl.kernel,
        out_type=jax.ShapeDtypeStruct((bt, d), jnp.float32),
        mesh=mesh,
        scratch_types=[
            pltpu.VMEM((rpw, 128), jnp.int32),
            pltpu.VMEM((128, d), jnp.float32),
            pltpu.VMEM((128, d), jnp.float32),
            pltpu.VMEM((16,), jnp.float32),
            pltpu.SemaphoreType.DMA,
            pltpu.SemaphoreType.DMA,
        ],
        compiler_params=pltpu.CompilerParams(use_tc_tiling_on_sc=False),
    )
    def k(table_hbm, idx_hbm, sinv_hbm, out_hbm,
          idx_v, buf_a, buf_b, sinv_v, sem_a, sem_b):
        wid = lax.axis_index("s") * nc + lax.axis_index("c")
        row0 = wid * rpw
        pltpu.sync_copy(idx_hbm.at[wid], idx_v)
        pltpu.sync_copy(sinv_hbm, sinv_v)
        s = sinv_v[...]

        def start(buf, sem, j):
            pltpu.async_copy(table_hbm.at[idx_v.at[j]], buf, sem)

        def finish(buf, sem, j):
            pltpu.make_async_copy(table_hbm.at[idx_v.at[j]], buf, sem).wait()

            def scale8(i, carry):
                r = i * 8
                for dr in range(8):
                    for h in range(d // 16):
                        sl = pl.ds(16 * h, 16)
                        buf[r + dr, sl] = buf[r + dr, sl] * s
                return carry

            lax.fori_loop(0, 16, scale8, 0)
            pltpu.sync_copy(buf, out_hbm.at[pl.ds((row0 + j) * 128, 128)])

        start(buf_a, sem_a, 0)
        start(buf_b, sem_b, 1)

        def pair(jp, carry):
            j0 = jp * 2
            finish(buf_a, sem_a, j0)
            start(buf_a, sem_a, j0 + 2)
            finish(buf_b, sem_b, j0 + 1)
            start(buf_b, sem_b, j0 + 3)
            return carry

        lax.fori_loop(0, rpw // 2 - 1, pair, 0)
        finish(buf_a, sem_a, rpw - 2)
        finish(buf_b, sem_b, rpw - 1)

    return k(table, idx3, sinv16)


def kernel(input, weight, u):
    v_rows, d = weight.shape
    blk = 20000
    assert v_rows % blk == 0
    u2 = u.reshape(v_rows // blk, 1, blk)
    sinv = _sigma_inv_tc(weight, u2, blk)
    sinv16 = jnp.broadcast_to(sinv.reshape(1), (16,))

    idx2 = input.reshape(-1).astype(jnp.int32).reshape(-1, 128)
    out = _gather_scale_sc(weight, idx2, sinv16)
    return out.reshape(input.shape + (d,))
